# baseline (speedup 1.0000x reference)
import jax
import jax.numpy as jnp
from jax.experimental import pallas as pl
from jax.experimental.pallas import tpu as pltpu

_M, _N, _DIM = 64, 128, 256
_NUM = _M * _N
_ALPHA = 0.3
_SIGMA = max(_M, _N) / 2.0


def _som_body(x_ref, w_ref, idx_ref, loc_ref, out_w_ref):
    w = w_ref[...]
    x = x_ref[...]
    e = x - w
    d2 = jnp.sum(e * e, axis=1, keepdims=True)
    dmin = jnp.min(d2)
    rows = jax.lax.broadcasted_iota(jnp.int32, (_NUM, 1), 0)
    bmu = jnp.min(jnp.where(d2 == dmin, rows, jnp.int32(_NUM)))
    bi = bmu // _N
    bj = bmu % _N
    idx_ref[0] = bmu
    loc_ref[0] = bi
    loc_ref[1] = bj

    ri = rows // _N
    rj = rows % _N
    di = (bi - ri).astype(jnp.float32)
    dj = (bj - rj).astype(jnp.float32)
    ld2 = di * di + dj * dj
    rate = 1.0 - rows.astype(jnp.float32) / float(_NUM)
    alpha_t = rate * _ALPHA
    sigma_t = rate * _SIGMA
    h = jnp.exp(-ld2 / (2.0 * sigma_t * sigma_t))
    out_w_ref[...] = w + (h * alpha_t) * e


def kernel(x, weights, locations):
    del locations
    x2 = x.reshape(1, _DIM)
    bmu_idx, bmu_loc, new_w = pl.pallas_call(
        _som_body,
        out_shape=(
            jax.ShapeDtypeStruct((1,), jnp.int32),
            jax.ShapeDtypeStruct((2,), jnp.int32),
            jax.ShapeDtypeStruct((_NUM, _DIM), jnp.float32),
        ),
        in_specs=[
            pl.BlockSpec(memory_space=pltpu.VMEM),
            pl.BlockSpec(memory_space=pltpu.VMEM),
        ],
        out_specs=(
            pl.BlockSpec(memory_space=pltpu.SMEM),
            pl.BlockSpec(memory_space=pltpu.SMEM),
            pl.BlockSpec(memory_space=pltpu.VMEM),
        ),
    )(x2, weights)
    return bmu_idx[0], bmu_loc, new_w
